# baseline (speedup 1.0000x reference)
import jax
import jax.numpy as jnp
from jax import lax
from jax.experimental import pallas as pl
from jax.experimental.pallas import tpu as pltpu
from jax.experimental.pallas import tpu_sc as plsc

EMBED = 64
NC = 2
NS = 16
NW = NC * NS
CHUNK = 128


def _gather_body(idx_hbm, emb_hbm, out_hbm, idx_v, rows_v, sem):
    wid = lax.axis_index("s") * NC + lax.axis_index("c")
    n_chunks = idx_v.shape[0]
    rows_per_w = n_chunks * CHUNK
    pltpu.sync_copy(idx_hbm.at[pl.ds(wid * n_chunks, n_chunks)], idx_v)
    copies = [
        pltpu.async_copy(
            emb_hbm.at[idx_v.at[j]],
            rows_v.at[pl.ds(j * CHUNK, CHUNK)],
            sem,
        )
        for j in range(n_chunks)
    ]
    for c in copies:
        c.wait()
    pltpu.sync_copy(rows_v, out_hbm.at[pl.ds(wid * rows_per_w, rows_per_w)])


def _sc_gather(idx2d, emb):
    n_total = idx2d.shape[0]
    n_per_w = n_total // NW
    return pl.kernel(
        _gather_body,
        out_type=jax.ShapeDtypeStruct((n_total * CHUNK, EMBED), jnp.float32),
        mesh=plsc.VectorSubcoreMesh(core_axis_name="c", subcore_axis_name="s"),
        scratch_types=[
            pltpu.VMEM((n_per_w, CHUNK), jnp.int32),
            pltpu.VMEM((n_per_w * CHUNK, EMBED), jnp.float32),
            pltpu.SemaphoreType.DMA,
        ],
        compiler_params=pltpu.CompilerParams(use_tc_tiling_on_sc=False),
    )(idx2d, emb)


def _mlp_body(e_ref, w1t_ref, b1_ref, w2t_ref, b2_ref, o_ref):
    h = jnp.dot(e_ref[...], w1t_ref[...], preferred_element_type=jnp.float32)
    h = jnp.maximum(h + b1_ref[...], 0.0)
    o = jnp.dot(h, w2t_ref[...], preferred_element_type=jnp.float32)
    o_ref[...] = jax.nn.sigmoid(o + b2_ref[...])


def _mlp(e, w1t, b1, w2t, b2, block_b):
    B, F = e.shape
    return pl.pallas_call(
        _mlp_body,
        grid=(B // block_b,),
        in_specs=[
            pl.BlockSpec((block_b, F), lambda i: (i, 0)),
            pl.BlockSpec((F, F), lambda i: (0, 0)),
            pl.BlockSpec((1, F), lambda i: (0, 0)),
            pl.BlockSpec((F, 1), lambda i: (0, 0)),
            pl.BlockSpec((1, 1), lambda i: (0, 0)),
        ],
        out_specs=pl.BlockSpec((block_b, 1), lambda i: (i, 0)),
        out_shape=jax.ShapeDtypeStruct((B, 1), jnp.float32),
    )(e, w1t, b1, w2t, b2)


def kernel(x, emb, W1, b1, W2, b2):
    B = x.shape[0]
    idx2d = x.reshape(-1, CHUNK)
    e_rows = _sc_gather(idx2d, emb)
    e = e_rows.reshape(B, 2 * EMBED)
    return _mlp(
        e,
        W1.T,
        b1.reshape(1, -1),
        W2.T,
        b2.reshape(1, 1),
        block_b=2048,
    )
